# Initial kernel scaffold; baseline (speedup 1.0000x reference)
import jax
import jax.numpy as jnp
from jax.experimental import pallas as pl


def kernel(observes, cidx, pidx, pose, intrinsics, points):
    raise NotImplementedError("write your pallas kernel here")


# speedup vs baseline: 3.5304x; 3.5304x over previous
import functools

import jax
import jax.numpy as jnp
from jax import lax
from jax.experimental import pallas as pl
from jax.experimental.pallas import tpu as pltpu
from jax.experimental.pallas import tpu_sc as plsc

NC = 2
NS = 16
NW = NC * NS
L = 16
CH = 128


def _fast_rsqrt(s):
    i = plsc.bitcast(s, jnp.int32)
    i = jnp.int32(0x5F3759DF) - lax.shift_right_logical(i, 1)
    y = plsc.bitcast(i, jnp.float32)
    for _ in range(3):
        y = y * (1.5 - 0.5 * s * y * y)
    return y


def _make_sc_kernel(npad, n_cam):
    nwo = npad // NW
    n_chunks = nwo // CH
    mesh = plsc.VectorSubcoreMesh(core_axis_name="c", subcore_axis_name="s")

    @functools.partial(
        pl.kernel,
        mesh=mesh,
        compiler_params=pltpu.CompilerParams(needs_layout_passes=False,
                                             use_tc_tiling_on_sc=False),
        out_type=jax.ShapeDtypeStruct((2 * npad,), jnp.float32),
        scratch_types=[
            pltpu.VMEM((7 * n_cam,), jnp.float32),
            pltpu.VMEM((3 * n_cam,), jnp.float32),
            pltpu.VMEM((CH,), jnp.int32),
            pltpu.VMEM((CH,), jnp.int32),
            pltpu.VMEM((2 * CH,), jnp.float32),
            pltpu.VMEM((CH, 3), jnp.float32),
            pltpu.VMEM((2 * CH,), jnp.float32),
            pltpu.SemaphoreType.DMA,
        ],
    )
    def sc_kernel(obs_hbm, cidx_hbm, pidx_hbm, pose_hbm, intr_hbm, pts_hbm,
                  out_hbm, pose_v, intr_v, cidx_v, pidx_v, obs_v, pts_v,
                  out_v, sem):
        wid = lax.axis_index("s") * NC + lax.axis_index("c")
        iota = lax.iota(jnp.int32, L)

        pltpu.sync_copy(pose_hbm, pose_v)
        pltpu.sync_copy(intr_hbm, intr_v)

        def norm_body(i, carry):
            b7 = (iota + i * L) * 7
            qw = plsc.load_gather(pose_v, [b7])
            qx = plsc.load_gather(pose_v, [b7 + 1])
            qy = plsc.load_gather(pose_v, [b7 + 2])
            qz = plsc.load_gather(pose_v, [b7 + 3])
            s = qw * qw + qx * qx + qy * qy + qz * qz
            y = _fast_rsqrt(s)
            plsc.store_scatter(pose_v, [b7], qw * y)
            plsc.store_scatter(pose_v, [b7 + 1], qx * y)
            plsc.store_scatter(pose_v, [b7 + 2], qy * y)
            plsc.store_scatter(pose_v, [b7 + 3], qz * y)
            return carry

        lax.fori_loop(0, n_cam // L, norm_body, 0)

        col0 = jnp.zeros((L,), jnp.int32)
        col1 = col0 + 1
        col2 = col0 + 2

        def chunk_body(j, carry):
            base = wid * nwo + j * CH
            pltpu.sync_copy(cidx_hbm.at[pl.ds(base, CH)], cidx_v)
            pltpu.sync_copy(pidx_hbm.at[pl.ds(base, CH)], pidx_v)
            pltpu.sync_copy(obs_hbm.at[pl.ds(2 * base, 2 * CH)], obs_v)
            pltpu.async_copy(pts_hbm.at[pidx_v], pts_v, sem).wait()

            for g in range(CH // L):
                o = g * L
                c16 = cidx_v[pl.ds(o, L)]
                b7 = c16 * 7
                b3 = c16 * 3
                qw = plsc.load_gather(pose_v, [b7])
                qx = plsc.load_gather(pose_v, [b7 + 1])
                qy = plsc.load_gather(pose_v, [b7 + 2])
                qz = plsc.load_gather(pose_v, [b7 + 3])
                tx = plsc.load_gather(pose_v, [b7 + 4])
                ty = plsc.load_gather(pose_v, [b7 + 5])
                tz = plsc.load_gather(pose_v, [b7 + 6])
                fo = plsc.load_gather(intr_v, [b3])
                k1 = plsc.load_gather(intr_v, [b3 + 1])
                k2 = plsc.load_gather(intr_v, [b3 + 2])
                rows = iota + o
                px = plsc.load_gather(pts_v, [rows, col0])
                py = plsc.load_gather(pts_v, [rows, col1])
                pz = plsc.load_gather(pts_v, [rows, col2])
                e2 = iota * 2 + (2 * o)
                ox = plsc.load_gather(obs_v, [e2])
                oy = plsc.load_gather(obs_v, [e2 + 1])

                ux = 2.0 * (qy * pz - qz * py)
                uy = 2.0 * (qz * px - qx * pz)
                uz = 2.0 * (qx * py - qy * px)
                cx = px + qw * ux + (qy * uz - qz * uy) + tx
                cy = py + qw * uy + (qz * ux - qx * uz) + ty
                cz = pz + qw * uz + (qx * uy - qy * ux) + tz
                nx = -cx / cz
                ny = -cy / cz
                r2 = nx * nx + ny * ny
                fd = fo * (1.0 + r2 * (k1 + k2 * r2))
                rx = fd * nx - ox
                ry = fd * ny - oy
                plsc.store_scatter(out_v, [e2], rx)
                plsc.store_scatter(out_v, [e2 + 1], ry)

            pltpu.sync_copy(out_v, out_hbm.at[pl.ds(2 * base, 2 * CH)])
            return carry

        lax.fori_loop(0, n_chunks, chunk_body, 0)

    return sc_kernel


def kernel(observes, cidx, pidx, pose, intrinsics, points):
    n = observes.shape[0]
    n_cam = pose.shape[0]
    block = NW * CH
    npad = ((n + block - 1) // block) * block
    pad = npad - n
    cidx_p = jnp.pad(cidx.astype(jnp.int32), (0, pad))
    pidx_p = jnp.pad(pidx.astype(jnp.int32), (0, pad))
    obs_p = jnp.pad(observes, ((0, pad), (0, 0))).reshape(-1)
    out = _make_sc_kernel(npad, n_cam)(
        obs_p, cidx_p, pidx_p, pose.reshape(-1), intrinsics.reshape(-1),
        points)
    return out.reshape(npad, 2)[:n]


# speedup vs baseline: 3.7302x; 1.0566x over previous
import functools

import jax
import jax.numpy as jnp
from jax import lax
from jax.experimental import pallas as pl
from jax.experimental.pallas import tpu as pltpu
from jax.experimental.pallas import tpu_sc as plsc

NC = 2
NS = 16
NW = NC * NS
L = 16
SUB = 128
NSUB = 4
CH = SUB * NSUB


def _fast_rsqrt(s):
    i = plsc.bitcast(s, jnp.int32)
    i = jnp.int32(0x5F3759DF) - lax.shift_right_logical(i, 1)
    y = plsc.bitcast(i, jnp.float32)
    for _ in range(3):
        y = y * (1.5 - 0.5 * s * y * y)
    return y


def _make_sc_kernel(npad, n_cam):
    nwo = npad // NW
    n_chunks = nwo // CH
    assert n_chunks % 2 == 0 and nwo % CH == 0
    mesh = plsc.VectorSubcoreMesh(core_axis_name="c", subcore_axis_name="s")

    buf_types = (
        [pltpu.VMEM((CH,), jnp.int32)]
        + [pltpu.VMEM((SUB,), jnp.int32)] * NSUB
        + [pltpu.VMEM((2 * CH,), jnp.float32)]
        + [pltpu.VMEM((SUB, 3), jnp.float32)] * NSUB
        + [pltpu.VMEM((2 * CH,), jnp.float32)]
        + [pltpu.SemaphoreType.DMA]
    )

    @functools.partial(
        pl.kernel,
        mesh=mesh,
        compiler_params=pltpu.CompilerParams(needs_layout_passes=False,
                                             use_tc_tiling_on_sc=False),
        out_type=jax.ShapeDtypeStruct((2 * npad,), jnp.float32),
        scratch_types=[
            pltpu.VMEM((7 * n_cam,), jnp.float32),
            pltpu.VMEM((3 * n_cam,), jnp.float32),
        ] + buf_types,
    )
    def sc_kernel(obs_hbm, cidx_hbm, pidx_hbm, pose_hbm, intr_hbm, pts_hbm,
                  out_hbm, pose_v, intr_v, *bufs):
        cidx_v = bufs[0]
        pidx_v = bufs[1:1 + NSUB]
        obs_v = bufs[1 + NSUB]
        pts_v = bufs[2 + NSUB:2 + 2 * NSUB]
        out_v = bufs[2 + 2 * NSUB]
        sem = bufs[3 + 2 * NSUB]

        wid = lax.axis_index("s") * NC + lax.axis_index("c")
        iota = lax.iota(jnp.int32, L)
        iota2 = iota * 2

        pltpu.sync_copy(pose_hbm, pose_v)
        pltpu.sync_copy(intr_hbm, intr_v)

        def _norm(i, carry):
            b7 = (iota + i * L) * 7
            qw = plsc.load_gather(pose_v, [b7])
            qx = plsc.load_gather(pose_v, [b7 + 1])
            qy = plsc.load_gather(pose_v, [b7 + 2])
            qz = plsc.load_gather(pose_v, [b7 + 3])
            s = qw * qw + qx * qx + qy * qy + qz * qz
            y = _fast_rsqrt(s)
            plsc.store_scatter(pose_v, [b7], qw * y)
            plsc.store_scatter(pose_v, [b7 + 1], qx * y)
            plsc.store_scatter(pose_v, [b7 + 2], qy * y)
            plsc.store_scatter(pose_v, [b7 + 3], qz * y)
            return carry

        lax.fori_loop(0, n_cam // L, _norm, 0)

        def chunk_body(j, carry):
            base = wid * nwo + j * CH
            pltpu.sync_copy(cidx_hbm.at[pl.ds(base, CH)], cidx_v)
            for k in range(NSUB):
                pltpu.sync_copy(pidx_hbm.at[pl.ds(base + k * SUB, SUB)],
                                pidx_v[k])
            pltpu.sync_copy(obs_hbm.at[pl.ds(2 * base, 2 * CH)], obs_v)
            for k in range(NSUB):
                pltpu.async_copy(pts_hbm.at[pidx_v[k]], pts_v[k], sem).wait()

            for k in range(NSUB):
                pts_b = pts_v[k]

                def _grp(g, carry2, k=k, pts_b=pts_b):
                    o = k * SUB + g * L
                    c16 = cidx_v[pl.ds(o, L)]
                    b7 = c16 * 7
                    b3 = c16 * 3
                    qw = plsc.load_gather(pose_v, [b7])
                    qx = plsc.load_gather(pose_v, [b7 + 1])
                    qy = plsc.load_gather(pose_v, [b7 + 2])
                    qz = plsc.load_gather(pose_v, [b7 + 3])
                    tx = plsc.load_gather(pose_v, [b7 + 4])
                    ty = plsc.load_gather(pose_v, [b7 + 5])
                    tz = plsc.load_gather(pose_v, [b7 + 6])
                    fo = plsc.load_gather(intr_v, [b3])
                    k1 = plsc.load_gather(intr_v, [b3 + 1])
                    k2 = plsc.load_gather(intr_v, [b3 + 2])
                    rows = iota + g * L
                    col0 = jnp.zeros((L,), jnp.int32)
                    px = plsc.load_gather(pts_b, [rows, col0])
                    py = plsc.load_gather(pts_b, [rows, col0 + 1])
                    pz = plsc.load_gather(pts_b, [rows, col0 + 2])
                    e2 = iota2 + 2 * o
                    ox = plsc.load_gather(obs_v, [e2])
                    oy = plsc.load_gather(obs_v, [e2 + 1])

                    ux = 2.0 * (qy * pz - qz * py)
                    uy = 2.0 * (qz * px - qx * pz)
                    uz = 2.0 * (qx * py - qy * px)
                    cx = px + qw * ux + (qy * uz - qz * uy) + tx
                    cy = py + qw * uy + (qz * ux - qx * uz) + ty
                    cz = pz + qw * uz + (qx * uy - qy * ux) + tz
                    nx = -cx / cz
                    ny = -cy / cz
                    r2 = nx * nx + ny * ny
                    fd = fo * (1.0 + r2 * (k1 + k2 * r2))
                    plsc.store_scatter(out_v, [e2], fd * nx - ox)
                    plsc.store_scatter(out_v, [e2 + 1], fd * ny - oy)
                    return carry2

                lax.fori_loop(0, SUB // L, _grp, 0)

            pltpu.sync_copy(out_v, out_hbm.at[pl.ds(2 * base, 2 * CH)])
            return carry

        lax.fori_loop(0, n_chunks, chunk_body, 0)

    return sc_kernel


def kernel(observes, cidx, pidx, pose, intrinsics, points):
    n = observes.shape[0]
    n_cam = pose.shape[0]
    block = NW * CH * 2
    npad = ((n + block - 1) // block) * block
    pad = npad - n
    cidx_p = jnp.pad(cidx.astype(jnp.int32), (0, pad))
    pidx_p = jnp.pad(pidx.astype(jnp.int32), (0, pad))
    obs_p = jnp.pad(observes, ((0, pad), (0, 0))).reshape(-1)
    out = _make_sc_kernel(npad, n_cam)(
        obs_p, cidx_p, pidx_p, pose.reshape(-1), intrinsics.reshape(-1),
        points)
    return out.reshape(npad, 2)[:n]
